# baseline (speedup 1.0000x reference)
import functools

import jax
import jax.numpy as jnp
from jax import lax
from jax.experimental import pallas as pl
from jax.experimental.pallas import tpu as pltpu
from jax.experimental.pallas import tpu_sc as plsc

DIM = 16
BATCH = 4096
NC = 2
NS = 16
NW = NC * NS
UPW = BATCH // NW
IPS = BATCH // NS
ICH = IPS // 128


@functools.partial(
    pl.kernel,
    mesh=plsc.VectorSubcoreMesh(core_axis_name="c", subcore_axis_name="s"),
    out_type=jax.ShapeDtypeStruct((BATCH,), jnp.float32),
    compiler_params=pltpu.CompilerParams(
        needs_layout_passes=False, use_tc_tiling_on_sc=False),
    scratch_types=[
        pltpu.VMEM((UPW,), jnp.int32),
        pltpu.VMEM((ICH, 128), jnp.int32),
        pltpu.VMEM((UPW, DIM), jnp.float32),
        pltpu.VMEM((ICH, 128, DIM), jnp.float32),
        pltpu.VMEM((DIM,), jnp.float32),
        pltpu.VMEM((NS, DIM), jnp.float32),
        pltpu.VMEM((UPW,), jnp.float32),
        pltpu.VMEM((16, 16), jnp.float32),
        pltpu.VMEM_SHARED((NS, DIM), jnp.float32),
        pltpu.SemaphoreType.DMA,
        pltpu.SemaphoreType.DMA,
    ],
)
def _mf_kernel(users_hbm, items_hbm, u_hbm, v_hbm, out_hbm,
               uidx, iidx, urows, vrows, part, allparts, outv, tile, shared,
               sem_u, sem_i):
    c = lax.axis_index("c")
    s = lax.axis_index("s")
    wid = s * NC + c
    ubase = wid * UPW
    ibase = s * IPS

    pltpu.sync_copy(users_hbm.at[pl.ds(ubase, UPW)], uidx)
    for j in range(ICH):
        pltpu.sync_copy(items_hbm.at[pl.ds(ibase + j * 128, 128)], iidx.at[j])

    ucp = pltpu.async_copy(u_hbm.at[uidx], urows, sem_u)
    icps = [pltpu.async_copy(v_hbm.at[iidx.at[j]], vrows.at[j], sem_i)
            for j in range(ICH)]
    for cp in icps:
        cp.wait()

    def isum_body(k, acc):
        j = k // 128
        kk = k - j * 128
        return acc + vrows[j, kk, :]
    acc = lax.fori_loop(0, IPS, isum_body, jnp.zeros((DIM,), jnp.float32))
    part[...] = acc

    pltpu.sync_copy(part, shared.at[s])
    plsc.subcore_barrier()
    pltpu.sync_copy(shared, allparts)
    svec = jnp.zeros((DIM,), jnp.float32)
    for t in range(NS):
        svec = svec + allparts[t, :]

    ucp.wait()

    lane = lax.iota(jnp.int32, 16)

    def dot_group(g, sv):
        for t in range(16):
            p = urows[g * 16 + t, :] * sv
            plsc.store_scatter(tile, [lane, jnp.full((16,), t, jnp.int32)], p)
        ovec = tile[0, :]
        for d in range(1, 16):
            ovec = ovec + tile[d, :]
        outv[pl.ds(g * 16, 16)] = ovec
        return sv
    lax.fori_loop(0, UPW // 16, dot_group, svec)

    pltpu.sync_copy(outv, out_hbm.at[pl.ds(ubase, UPW)])


def kernel(users, items, U, V):
    return _mf_kernel(users.astype(jnp.int32), items.astype(jnp.int32), U, V)
